# baseline (speedup 1.0000x reference)
import functools

import jax
import jax.numpy as jnp
from jax import lax
from jax.experimental import pallas as pl
from jax.experimental.pallas import tpu as pltpu
from jax.experimental.pallas import tpu_sc as plsc

_TN = 256
_TNC = 512
_CHUNK = 128

_BIG = 1e30


def _nn_body(u_ref, kT_ref, idx0_ref, idx1_ref, idx2_ref,
             w0_ref, w1_ref, w2_ref, *, m):
    b = pl.program_id(0)
    u = u_ref[...]
    kT = kT_ref[...]
    G = jnp.dot(u, kT, preferred_element_type=jnp.float32,
                precision=lax.Precision.HIGHEST)
    unorm = jnp.sum(u * u, axis=1, keepdims=True)
    knorm = jnp.sum(kT * kT, axis=0, keepdims=True)
    d2 = jnp.maximum((unorm + knorm) - 2.0 * G, 0.0)

    iota = lax.broadcasted_iota(jnp.int32, d2.shape, 1)
    D = d2
    mins, idxs = [], []
    for s in range(3):
        mn = jnp.min(D, axis=1, keepdims=True)
        cand = jnp.where(D == mn, iota, m)
        amn = jnp.min(cand, axis=1, keepdims=True)
        mins.append(mn)
        idxs.append(amn)
        if s < 2:
            D = jnp.where(cand == amn, _BIG, D)

    ws = [1.0 / (jnp.sqrt(mn) + 1e-8) for mn in mins]
    norm = ws[0] + ws[1] + ws[2]
    base = b * m
    idx0_ref[...] = idxs[0] + base
    idx1_ref[...] = idxs[1] + base
    idx2_ref[...] = idxs[2] + base
    w0_ref[...] = ws[0] / norm
    w1_ref[...] = ws[1] / norm
    w2_ref[...] = ws[2] / norm


def _three_nn(unknown8, knownT8):
    B, n, _ = unknown8.shape
    m = knownT8.shape[2]
    grid = (B, n // _TN)
    iout = jax.ShapeDtypeStruct((B, n, 1), jnp.int32)
    fout = jax.ShapeDtypeStruct((B, n, 1), jnp.float32)
    nspec = pl.BlockSpec((None, _TN, 1), lambda b, i: (b, i, 0))
    return pl.pallas_call(
        functools.partial(_nn_body, m=m),
        grid=grid,
        in_specs=[
            pl.BlockSpec((None, _TN, 8), lambda b, i: (b, i, 0)),
            pl.BlockSpec((None, 8, m), lambda b, i: (b, 0, 0)),
        ],
        out_specs=[nspec] * 6,
        out_shape=[iout, iout, iout, fout, fout, fout],
    )(unknown8, knownT8)


def _sc_gather_serial(table, idx0, idx1, idx2):
    N = idx0.shape[0]
    C2 = table.shape[1]
    info = plsc.get_sparse_core_info()
    nw = info.num_cores * info.num_subcores
    per_w = N // nw
    nchunk = per_w // _CHUNK
    mesh = plsc.VectorSubcoreMesh(core_axis_name="c", subcore_axis_name="s")
    gout = jax.ShapeDtypeStruct((N, C2), jnp.float32)

    @functools.partial(
        pl.kernel, mesh=mesh,
        out_type=(gout, gout, gout),
        scratch_types=[
            pltpu.VMEM((_CHUNK,), jnp.int32),
            pltpu.VMEM((_CHUNK, C2), jnp.float32),
            pltpu.SemaphoreType.DMA,
        ],
    )
    def gather_kernel(table_hbm, i0_hbm, i1_hbm, i2_hbm,
                      g0_hbm, g1_hbm, g2_hbm, idx_v, rows_v, sem):
        wid = lax.axis_index("s") * info.num_cores + lax.axis_index("c")
        base = wid * per_w
        for idx_hbm, g_hbm in ((i0_hbm, g0_hbm), (i1_hbm, g1_hbm),
                               (i2_hbm, g2_hbm)):
            for c in range(nchunk):
                off = base + c * _CHUNK
                pltpu.sync_copy(idx_hbm.at[pl.ds(off, _CHUNK)], idx_v)
                pltpu.async_copy(table_hbm.at[idx_v], rows_v, sem).wait()
                pltpu.sync_copy(rows_v, g_hbm.at[pl.ds(off, _CHUNK), :])

    return gather_kernel(table, idx0, idx1, idx2)


def _sc_gather(table, idx0, idx1, idx2):
    nrows, chunk = idx0.shape
    N = nrows * chunk
    C2 = table.shape[1]
    info = plsc.get_sparse_core_info()
    nw = info.num_cores * info.num_subcores
    per_w = N // nw
    nchunk = per_w // chunk
    ntask = 3 * nchunk
    nbuf = 4
    mesh = plsc.VectorSubcoreMesh(core_axis_name="c", subcore_axis_name="s")
    gout = jax.ShapeDtypeStruct((N, C2), jnp.float32)

    @functools.partial(
        pl.kernel, mesh=mesh,
        out_type=(gout, gout, gout),
        scratch_types=[
            pltpu.VMEM((ntask, chunk), jnp.int32),
            pltpu.VMEM((nbuf, chunk, C2), jnp.float32),
            [pltpu.SemaphoreType.DMA] * nbuf,
            [pltpu.SemaphoreType.DMA] * nbuf,
        ],
    )
    def gather_kernel(table_hbm, i0_hbm, i1_hbm, i2_hbm,
                      g0_hbm, g1_hbm, g2_hbm, idx_all, rows, gsems, wsems):
        wid = lax.axis_index("s") * info.num_cores + lax.axis_index("c")
        row0 = wid * nchunk
        for j, ih in enumerate((i0_hbm, i1_hbm, i2_hbm)):
            pltpu.sync_copy(ih.at[pl.ds(row0, nchunk), :],
                            idx_all.at[pl.ds(j * nchunk, nchunk), :])

        ghandles = [None] * nbuf

        def start_gather(t):
            buf = t % nbuf
            ghandles[buf] = pltpu.async_copy(
                table_hbm.at[idx_all.at[t]], rows.at[buf], gsems[buf])

        for t in range(nbuf):
            start_gather(t)
        gouts = (g0_hbm, g1_hbm, g2_hbm)
        for t in range(ntask):
            buf = t % nbuf
            ghandles[buf].wait()
            j, c = divmod(t, nchunk)
            off = wid * per_w + c * chunk
            wh = pltpu.async_copy(rows.at[buf],
                                  gouts[j].at[pl.ds(off, chunk), :],
                                  wsems[buf])
            wh.wait()
            if t + nbuf < ntask:
                start_gather(t + nbuf)

    return gather_kernel(table, idx0, idx1, idx2)


def _mlp_body(g0_ref, g1_ref, g2_ref, w0_ref, w1_ref, w2_ref,
              uf_ref, w1a_ref, w1b_ref, b1_ref, out_ref):
    interp = (w0_ref[...] * g0_ref[...] + w1_ref[...] * g1_ref[...]
              + w2_ref[...] * g2_ref[...])
    acc = lax.dot_general(w1a_ref[...], interp,
                          (((1,), (1,)), ((), ())),
                          preferred_element_type=jnp.float32)
    acc = acc + jnp.dot(w1b_ref[...], uf_ref[...],
                        preferred_element_type=jnp.float32)
    out_ref[...] = jnp.maximum(acc + b1_ref[...], 0.0)


def _mlp(g0, g1, g2, w0, w1, w2, unknow_feats, W1a, W1b, b1c):
    B, C1, n = unknow_feats.shape
    C2 = g0.shape[2]
    Co = W1a.shape[0]
    grid = (B, n // _TNC)
    gspec = pl.BlockSpec((None, _TNC, C2), lambda b, i: (b, i, 0))
    wspec = pl.BlockSpec((None, _TNC, 1), lambda b, i: (b, i, 0))
    return pl.pallas_call(
        _mlp_body,
        grid=grid,
        in_specs=[
            gspec, gspec, gspec, wspec, wspec, wspec,
            pl.BlockSpec((None, C1, _TNC), lambda b, i: (b, 0, i)),
            pl.BlockSpec((Co, C2), lambda b, i: (0, 0)),
            pl.BlockSpec((Co, C1), lambda b, i: (0, 0)),
            pl.BlockSpec((Co, 1), lambda b, i: (0, 0)),
        ],
        out_specs=pl.BlockSpec((None, Co, _TNC), lambda b, i: (b, 0, i)),
        out_shape=jax.ShapeDtypeStruct((B, Co, n), jnp.float32),
    )(g0, g1, g2, w0, w1, w2, unknow_feats, W1a, W1b, b1c)


def kernel(unknown, known, unknow_feats, known_feats, W1, b1):
    B, n, _ = unknown.shape
    m = known.shape[1]
    C2 = known_feats.shape[1]

    pad_u = jnp.zeros((B, n, 5), jnp.float32)
    unknown8 = jnp.concatenate([unknown, pad_u], axis=2)
    pad_k = jnp.zeros((B, 5, m), jnp.float32)
    knownT8 = jnp.concatenate([jnp.transpose(known, (0, 2, 1)), pad_k],
                              axis=1)
    table = jnp.transpose(known_feats, (0, 2, 1)).reshape(B * m, C2)

    idx0, idx1, idx2, w0, w1, w2 = _three_nn(unknown8, knownT8)

    nr = (B * n) // _CHUNK
    g0, g1, g2 = _sc_gather(table, idx0.reshape(nr, _CHUNK),
                            idx1.reshape(nr, _CHUNK), idx2.reshape(nr, _CHUNK))
    g0 = g0.reshape(B, n, C2)
    g1 = g1.reshape(B, n, C2)
    g2 = g2.reshape(B, n, C2)

    W1a = W1[:, :C2]
    W1b = W1[:, C2:]
    b1c = b1.reshape(-1, 1)
    return _mlp(g0, g1, g2, w0, w1, w2, unknow_feats, W1a, W1b, b1c)


# speedup vs baseline: 1.3245x; 1.3245x over previous
import functools

import jax
import jax.numpy as jnp
from jax import lax
from jax.experimental import pallas as pl
from jax.experimental.pallas import tpu as pltpu
from jax.experimental.pallas import tpu_sc as plsc

_TN = 256
_TNC = 512
_CHUNK = 128

_BIG = 1e30


def _nn_body(u_ref, kT_ref, idx0_ref, idx1_ref, idx2_ref,
             w0_ref, w1_ref, w2_ref, *, m):
    b = pl.program_id(0)
    u = u_ref[...]
    kT = kT_ref[...]
    ux, uy, uz = u[:, 0:1], u[:, 1:2], u[:, 2:3]
    kx, ky, kz = kT[0:1, :], kT[1:2, :], kT[2:3, :]
    d2 = (ux - kx) ** 2 + (uy - ky) ** 2 + (uz - kz) ** 2

    iota = lax.broadcasted_iota(jnp.int32, d2.shape, 1)
    D = d2
    mins, idxs = [], []
    for s in range(3):
        mn = jnp.min(D, axis=1, keepdims=True)
        cand = jnp.where(D == mn, iota, m)
        amn = jnp.min(cand, axis=1, keepdims=True)
        mins.append(mn)
        idxs.append(amn)
        if s < 2:
            D = jnp.where(cand == amn, _BIG, D)

    ws = [1.0 / (jnp.sqrt(mn) + 1e-8) for mn in mins]
    norm = ws[0] + ws[1] + ws[2]
    base = b * m
    idx0_ref[...] = idxs[0] + base
    idx1_ref[...] = idxs[1] + base
    idx2_ref[...] = idxs[2] + base
    w0_ref[...] = ws[0] / norm
    w1_ref[...] = ws[1] / norm
    w2_ref[...] = ws[2] / norm


def _three_nn(unknown8, knownT8):
    B, n, _ = unknown8.shape
    m = knownT8.shape[2]
    grid = (B, n // _TN)
    iout = jax.ShapeDtypeStruct((B, n, 1), jnp.int32)
    fout = jax.ShapeDtypeStruct((B, n, 1), jnp.float32)
    nspec = pl.BlockSpec((None, _TN, 1), lambda b, i: (b, i, 0))
    return pl.pallas_call(
        functools.partial(_nn_body, m=m),
        grid=grid,
        in_specs=[
            pl.BlockSpec((None, _TN, 3), lambda b, i: (b, i, 0)),
            pl.BlockSpec((None, 3, m), lambda b, i: (b, 0, 0)),
        ],
        out_specs=[nspec] * 6,
        out_shape=[iout, iout, iout, fout, fout, fout],
    )(unknown8, knownT8)


def _sc_gather_serial(table, idx0, idx1, idx2):
    N = idx0.shape[0]
    C2 = table.shape[1]
    info = plsc.get_sparse_core_info()
    nw = info.num_cores * info.num_subcores
    per_w = N // nw
    nchunk = per_w // _CHUNK
    mesh = plsc.VectorSubcoreMesh(core_axis_name="c", subcore_axis_name="s")
    gout = jax.ShapeDtypeStruct((N, C2), jnp.float32)

    @functools.partial(
        pl.kernel, mesh=mesh,
        out_type=(gout, gout, gout),
        scratch_types=[
            pltpu.VMEM((_CHUNK,), jnp.int32),
            pltpu.VMEM((_CHUNK, C2), jnp.float32),
            pltpu.SemaphoreType.DMA,
        ],
    )
    def gather_kernel(table_hbm, i0_hbm, i1_hbm, i2_hbm,
                      g0_hbm, g1_hbm, g2_hbm, idx_v, rows_v, sem):
        wid = lax.axis_index("s") * info.num_cores + lax.axis_index("c")
        base = wid * per_w
        for idx_hbm, g_hbm in ((i0_hbm, g0_hbm), (i1_hbm, g1_hbm),
                               (i2_hbm, g2_hbm)):
            for c in range(nchunk):
                off = base + c * _CHUNK
                pltpu.sync_copy(idx_hbm.at[pl.ds(off, _CHUNK)], idx_v)
                pltpu.async_copy(table_hbm.at[idx_v], rows_v, sem).wait()
                pltpu.sync_copy(rows_v, g_hbm.at[pl.ds(off, _CHUNK), :])

    return gather_kernel(table, idx0, idx1, idx2)


def _sc_gather(table, idx0, idx1, idx2):
    nrows, chunk = idx0.shape
    N = nrows * chunk
    C2 = table.shape[1]
    info = plsc.get_sparse_core_info()
    nw = info.num_cores * info.num_subcores
    per_w = N // nw
    nchunk = per_w // chunk
    ntask = 3 * nchunk
    nbuf = 4
    mesh = plsc.VectorSubcoreMesh(core_axis_name="c", subcore_axis_name="s")
    gout = jax.ShapeDtypeStruct((N, C2), jnp.float32)

    @functools.partial(
        pl.kernel, mesh=mesh,
        out_type=(gout, gout, gout),
        scratch_types=[
            pltpu.VMEM((ntask, chunk), jnp.int32),
            pltpu.VMEM((nbuf, chunk, C2), jnp.float32),
            [pltpu.SemaphoreType.DMA] * nbuf,
            [pltpu.SemaphoreType.DMA] * nbuf,
        ],
    )
    def gather_kernel(table_hbm, i0_hbm, i1_hbm, i2_hbm,
                      g0_hbm, g1_hbm, g2_hbm, idx_all, rows, gsems, wsems):
        wid = lax.axis_index("s") * info.num_cores + lax.axis_index("c")
        row0 = wid * nchunk
        for j, ih in enumerate((i0_hbm, i1_hbm, i2_hbm)):
            pltpu.sync_copy(ih.at[pl.ds(row0, nchunk), :],
                            idx_all.at[pl.ds(j * nchunk, nchunk), :])

        ghandles = [None] * nbuf

        def start_gather(t):
            buf = t % nbuf
            ghandles[buf] = pltpu.async_copy(
                table_hbm.at[idx_all.at[t]], rows.at[buf], gsems[buf])

        for t in range(nbuf):
            start_gather(t)
        gouts = (g0_hbm, g1_hbm, g2_hbm)
        for t in range(ntask):
            buf = t % nbuf
            ghandles[buf].wait()
            j, c = divmod(t, nchunk)
            off = wid * per_w + c * chunk
            wh = pltpu.async_copy(rows.at[buf],
                                  gouts[j].at[pl.ds(off, chunk), :],
                                  wsems[buf])
            wh.wait()
            if t + nbuf < ntask:
                start_gather(t + nbuf)

    return gather_kernel(table, idx0, idx1, idx2)


def _mlp_body(g0_ref, g1_ref, g2_ref, w0_ref, w1_ref, w2_ref,
              uf_ref, w1a_ref, w1b_ref, b1_ref, out_ref):
    interp = (w0_ref[...] * g0_ref[...] + w1_ref[...] * g1_ref[...]
              + w2_ref[...] * g2_ref[...])
    acc = lax.dot_general(w1a_ref[...], interp,
                          (((1,), (1,)), ((), ())),
                          preferred_element_type=jnp.float32)
    acc = acc + jnp.dot(w1b_ref[...], uf_ref[...],
                        preferred_element_type=jnp.float32)
    out_ref[...] = jnp.maximum(acc + b1_ref[...], 0.0)


def _mlp(g0, g1, g2, w0, w1, w2, unknow_feats, W1a, W1b, b1c):
    B, C1, n = unknow_feats.shape
    C2 = g0.shape[2]
    Co = W1a.shape[0]
    grid = (B, n // _TNC)
    gspec = pl.BlockSpec((None, _TNC, C2), lambda b, i: (b, i, 0))
    wspec = pl.BlockSpec((None, _TNC, 1), lambda b, i: (b, i, 0))
    return pl.pallas_call(
        _mlp_body,
        grid=grid,
        in_specs=[
            gspec, gspec, gspec, wspec, wspec, wspec,
            pl.BlockSpec((None, C1, _TNC), lambda b, i: (b, 0, i)),
            pl.BlockSpec((Co, C2), lambda b, i: (0, 0)),
            pl.BlockSpec((Co, C1), lambda b, i: (0, 0)),
            pl.BlockSpec((Co, 1), lambda b, i: (0, 0)),
        ],
        out_specs=pl.BlockSpec((None, Co, _TNC), lambda b, i: (b, 0, i)),
        out_shape=jax.ShapeDtypeStruct((B, Co, n), jnp.float32),
    )(g0, g1, g2, w0, w1, w2, unknow_feats, W1a, W1b, b1c)


def kernel(unknown, known, unknow_feats, known_feats, W1, b1):
    B, n, _ = unknown.shape
    m = known.shape[1]
    C2 = known_feats.shape[1]

    knownT = jnp.transpose(known, (0, 2, 1))
    table = jnp.transpose(known_feats, (0, 2, 1)).reshape(B * m, C2)

    idx0, idx1, idx2, w0, w1, w2 = _three_nn(unknown, knownT)

    nr = (B * n) // _CHUNK
    g0, g1, g2 = _sc_gather(table, idx0.reshape(nr, _CHUNK),
                            idx1.reshape(nr, _CHUNK), idx2.reshape(nr, _CHUNK))
    g0 = g0.reshape(B, n, C2)
    g1 = g1.reshape(B, n, C2)
    g2 = g2.reshape(B, n, C2)

    W1a = W1[:, :C2]
    W1b = W1[:, C2:]
    b1c = b1.reshape(-1, 1)
    return _mlp(g0, g1, g2, w0, w1, w2, unknow_feats, W1a, W1b, b1c)
